# baseline (speedup 1.0000x reference)
import jax
import jax.numpy as jnp
from jax import lax
from jax.experimental import pallas as pl
from jax.experimental.pallas import tpu as pltpu

_VMEM_LIMIT = 48 * 1024 * 1024


def _pick_tile(n: int, candidates) -> int:
    for c in candidates:
        if c <= n and n % c == 0:
            return c
    return n


def _edge_idx_kernel(src_ref, dst_ref, x2_ref, idx_ref):
    b = src_ref.shape[2]
    s = src_ref[0]
    lo = s & 127
    hi = s >> 7
    oh_lo = (lax.broadcasted_iota(jnp.int32, (128, b), 0) == lo
             ).astype(jnp.int8)
    y = jnp.dot(x2_ref[...], oh_lo,
                preferred_element_type=jnp.int32)
    hi_eq = lax.broadcasted_iota(jnp.int32, (64, b), 0) == hi
    lab = jnp.sum(jnp.where(hi_eq, y, 0), axis=0, keepdims=True)
    idx_ref[0] = dst_ref[0] * 64 + lab


def _prep_kernel(ci_ref, xc_ref, sb_ref, n2h_ref):
    tp = ci_ref.shape[0]
    c = ci_ref[...].astype(jnp.float32)
    oh = (lax.broadcasted_iota(jnp.int32, (tp, 64), 1) == xc_ref[...]
          ).astype(jnp.bfloat16)
    sb_ref[...] = jnp.concatenate([c.astype(jnp.bfloat16), oh], axis=1)
    n2h_ref[...] = (jnp.sum(c * c, axis=1, keepdims=True) + 1.0) * 0.5


def _match_kernel(ct_ref, call_ref, n2hc_ref, n2hr_ref, first_ref):
    tq = first_ref.shape[0]
    n = call_ref.shape[0]
    cb = 2048
    row0 = pl.program_id(0) * tq
    nblk = (row0 + tq + cb - 1) // cb

    ct = ct_ref[...]
    thrc = n2hc_ref[...] - 0.25

    def body(k, cur):
        c0 = k * cb
        g = lax.dot_general(ct, call_ref[pl.ds(c0, cb), :],
                            dimension_numbers=(((1,), (1,)), ((), ())),
                            preferred_element_type=jnp.float32)
        thr = thrc + n2hr_ref[:, pl.ds(c0, cb)]
        cj = c0 + lax.broadcasted_iota(jnp.int32, (tq, cb), 1)
        cand = jnp.min(jnp.where(g > thr, cj, n), axis=1, keepdims=True)
        return jnp.minimum(cur, cand)

    first_ref[...] = lax.fori_loop(
        0, nblk, body, jnp.full((tq, 1), n, jnp.int32))


def _colors_kernel(fc_ref, fr_ref, out_ref):
    tq = out_ref.shape[0]
    cb = 2048
    row0 = pl.program_id(0) * tq
    nblk = (row0 + tq + cb - 1) // cb
    fc = fc_ref[...]

    def body(k, acc):
        c0 = k * cb
        fr = fr_ref[:, pl.ds(c0, cb)]
        cj = c0 + lax.broadcasted_iota(jnp.int32, (tq, cb), 1)
        rep = fr == (c0 + lax.broadcasted_iota(jnp.int32, (1, cb), 1))
        counted = jnp.logical_and(rep, cj < fc)
        return acc + jnp.sum(counted.astype(jnp.int32), axis=1, keepdims=True)

    out_ref[...] = lax.fori_loop(
        0, nblk, body, jnp.zeros((tq, 1), jnp.int32))


def kernel(x_labels, edge_index):
    N = int(x_labels.shape[0])
    E = int(edge_index.shape[1])
    C = 64
    Cp = 128
    src, dst = edge_index[0], edge_index[1]
    x32 = x_labels.astype(jnp.int32)

    eb = 32768
    while E % eb:
        eb //= 2
    g = E // eb
    src3 = src.reshape(g, 1, eb)
    dst3 = dst.reshape(g, 1, eb)
    x2 = x32.reshape(C, Cp).astype(jnp.int8)

    e_idx = pl.pallas_call(
        _edge_idx_kernel,
        out_shape=jax.ShapeDtypeStruct((g, 1, eb), jnp.int32),
        grid=(g,),
        in_specs=[
            pl.BlockSpec((1, 1, eb), lambda i: (i, 0, 0)),
            pl.BlockSpec((1, 1, eb), lambda i: (i, 0, 0)),
            pl.BlockSpec((C, Cp), lambda i: (0, 0)),
        ],
        out_specs=pl.BlockSpec((1, 1, eb), lambda i: (i, 0, 0)),
        compiler_params=pltpu.CompilerParams(
            dimension_semantics=("parallel",),
            vmem_limit_bytes=_VMEM_LIMIT),
    )(src3, dst3, x2)

    flat = jnp.zeros((N * C,), jnp.int32)
    flat = flat.at[e_idx.reshape(E)].add(1)
    hist = flat.reshape(N, C)

    tp = _pick_tile(N, (1024, 512, 256, 128, 64, 32, 16, 8))
    sig_bf16, n2h = pl.pallas_call(
        _prep_kernel,
        out_shape=(jax.ShapeDtypeStruct((N, Cp), jnp.bfloat16),
                   jax.ShapeDtypeStruct((N, 1), jnp.float32)),
        grid=(N // tp,),
        in_specs=[pl.BlockSpec((tp, C), lambda i: (i, 0)),
                  pl.BlockSpec((tp, 1), lambda i: (i, 0))],
        out_specs=(pl.BlockSpec((tp, Cp), lambda i: (i, 0)),
                   pl.BlockSpec((tp, 1), lambda i: (i, 0))),
        compiler_params=pltpu.CompilerParams(
            dimension_semantics=("parallel",),
            vmem_limit_bytes=_VMEM_LIMIT),
    )(hist, x32.reshape(N, 1))

    tq = _pick_tile(N, (256, 128, 64, 32, 16, 8))
    first = pl.pallas_call(
        _match_kernel,
        out_shape=jax.ShapeDtypeStruct((N, 1), jnp.int32),
        grid=(N // tq,),
        in_specs=[
            pl.BlockSpec((tq, Cp), lambda i: (i, 0)),
            pl.BlockSpec((N, Cp), lambda i: (0, 0)),
            pl.BlockSpec((tq, 1), lambda i: (i, 0)),
            pl.BlockSpec((1, N), lambda i: (0, 0)),
        ],
        out_specs=pl.BlockSpec((tq, 1), lambda i: (i, 0)),
        compiler_params=pltpu.CompilerParams(
            dimension_semantics=("parallel",),
            vmem_limit_bytes=_VMEM_LIMIT),
    )(sig_bf16, sig_bf16, n2h, n2h.reshape(1, N))

    colors = pl.pallas_call(
        _colors_kernel,
        out_shape=jax.ShapeDtypeStruct((N, 1), jnp.int32),
        grid=(N // tq,),
        in_specs=[
            pl.BlockSpec((tq, 1), lambda i: (i, 0)),
            pl.BlockSpec((1, N), lambda i: (0, 0)),
        ],
        out_specs=pl.BlockSpec((tq, 1), lambda i: (i, 0)),
        compiler_params=pltpu.CompilerParams(
            dimension_semantics=("parallel",),
            vmem_limit_bytes=_VMEM_LIMIT),
    )(first, first.reshape(1, N))

    return colors[:, 0]
